# baseline (speedup 1.0000x reference)
import jax
import jax.numpy as jnp
import numpy as np
from jax import lax
from jax.experimental import pallas as pl
from jax.experimental.pallas import tpu as pltpu
from jax.experimental.pallas import tpu_sc as plsc

STRIDES = (8, 16, 32, 64, 128)
WINDOW = ((-1.0, 64.0), (64.0, 128.0), (128.0, 256.0), (256.0, 512.0),
          (512.0, 99999999.0))
HW = ((64, 64), (32, 32), (16, 16), (8, 8), (4, 4))
B = 8
M = 50
BIG = 99999999.0

NW = 32
LANES = 16
PW = 1376
P_PAD = NW * PW

LVL_PX = (1024, 256, 64, 16, 16)
LVL_OFF = (0, 1024, 1280, 1344, 1360)


def _build_coords():
    xf = np.zeros(P_PAD, np.float32)
    yf = np.zeros(P_PAD, np.float32)
    for w in range(NW):
        base = w * PW
        for li, (h, wd) in enumerate(HW):
            s = STRIDES[li]
            npx = LVL_PX[li]
            if li == 4 and w >= 8:
                continue
            f = w * npx + np.arange(npx)
            p = f % (h * wd)
            xf[base + LVL_OFF[li]: base + LVL_OFF[li] + npx] = \
                (p % wd) * s + s // 2
            yf[base + LVL_OFF[li]: base + LVL_OFF[li] + npx] = \
                (p // wd) * s + s // 2
    return xf, yf


_XF, _YF = _build_coords()


def _sqrt16(x):
    xi = lax.bitcast_convert_type(x, jnp.int32)
    yi = jnp.int32(0x5F3759DF) - (xi >> 1)
    y = lax.bitcast_convert_type(yi, jnp.float32)
    hx = x * 0.5
    for _ in range(4):
        y = y * (1.5 - hx * y * y)
    return x * y


def _sc_body(bbx_h, cls_h, xf_h, yf_h,
             k0_h, k1_h, k2_h, k3_h, k4_h,
             r0_h, r1_h, r2_h, r3_h, r4_h,
             n0_h, n1_h, n2_h, n3_h, n4_h,
             bbxv, clsv, x0v, y0v, x1v, y1v, sxv, syv,
             xv, yv, ksv, nsv, rgv):
    wid = lax.axis_index("s") * 2 + lax.axis_index("c")
    poff = wid * PW
    pltpu.sync_copy(bbx_h, bbxv)
    pltpu.sync_copy(cls_h, clsv)
    pltpu.sync_copy(xf_h.at[pl.ds(poff, PW)], xv)
    pltpu.sync_copy(yf_h.at[pl.ds(poff, PW)], yv)

    iota = lax.iota(jnp.int32, LANES)
    iota4 = iota * 4
    for v in range(B * M // LANES):
        idx = iota4 + (64 * v)
        gx0 = plsc.load_gather(bbxv, (idx,))
        gy0 = plsc.load_gather(bbxv, (idx + 1,))
        gx1 = plsc.load_gather(bbxv, (idx + 2,))
        gy1 = plsc.load_gather(bbxv, (idx + 3,))
        sl = pl.ds(v * LANES, LANES)
        x0v[sl] = gx0
        y0v[sl] = gy0
        x1v[sl] = gx1
        y1v[sl] = gy1
        sxv[sl] = gx0 + gx1
        syv[sl] = gy0 + gy1

    def chunk_body(i, soff, bb, wlo, whi, rad2):
        s = soff + i * LANES
        px = xv[pl.ds(s, LANES)]
        py = yv[pl.ds(s, LANES)]
        x2 = px + px
        y2 = py + py
        best_a = jnp.full((LANES,), BIG, jnp.float32)
        best_m = jnp.zeros((LANES,), jnp.int32)
        anyp = jnp.zeros((LANES,), jnp.bool_)
        for m in range(M):
            mi = bb + m
            bx0 = plsc.load_gather(x0v, (mi,))
            by0 = plsc.load_gather(y0v, (mi,))
            bx1 = plsc.load_gather(x1v, (mi,))
            by1 = plsc.load_gather(y1v, (mi,))
            bsx = plsc.load_gather(sxv, (mi,))
            bsy = plsc.load_gather(syv, (mi,))
            l = px - bx0
            t = py - by0
            r = bx1 - px
            b2 = by1 - py
            dmin = jnp.minimum(jnp.minimum(l, t), jnp.minimum(r, b2))
            dmax = jnp.maximum(jnp.maximum(l, t), jnp.maximum(r, b2))
            dc = jnp.maximum(jnp.abs(x2 - bsx), jnp.abs(y2 - bsy))
            pos = (dmin > 0.0) & (dc < rad2)
            if whi is not None:
                pos = pos & (dmax <= whi)
            if wlo is not None:
                pos = pos & (dmax >= wlo)
            area = (l + r) * (t + b2)
            am = jnp.where(pos, area, BIG)
            better = am < best_a
            best_a = jnp.where(better, am, best_a)
            best_m = jnp.where(better, jnp.int32(m), best_m)
            anyp = anyp | pos
        gi = bb + best_m
        gx0 = plsc.load_gather(x0v, (gi,))
        gy0 = plsc.load_gather(y0v, (gi,))
        gx1 = plsc.load_gather(x1v, (gi,))
        gy1 = plsc.load_gather(y1v, (gi,))
        gcls = plsc.load_gather(clsv, (gi,))
        l = px - gx0
        t = py - gy0
        r = gx1 - px
        b2 = gy1 - py
        lrmin = jnp.minimum(l, r)
        lrmax = jnp.maximum(l, r)
        tbmin = jnp.minimum(t, b2)
        tbmax = jnp.maximum(t, b2)
        ratio = jnp.maximum(lrmin * tbmin / (lrmax * tbmax + 1e-10), 0.0)
        cnt = _sqrt16(ratio + 1e-12)
        neg1 = jnp.full((LANES,), -1.0, jnp.float32)
        s4 = s * 4
        idx0 = iota4 + s4
        plsc.store_scatter(rgv, (idx0,), jnp.where(anyp, l, neg1))
        plsc.store_scatter(rgv, (idx0 + 1,), jnp.where(anyp, t, neg1))
        plsc.store_scatter(rgv, (idx0 + 2,), jnp.where(anyp, r, neg1))
        plsc.store_scatter(rgv, (idx0 + 3,), jnp.where(anyp, b2, neg1))
        nsv[pl.ds(s, LANES)] = jnp.where(anyp, cnt, neg1)
        ksv[pl.ds(s, LANES)] = jnp.where(
            anyp, gcls, jnp.zeros((LANES,), jnp.int32))
        return 0

    bb_main = jnp.full((LANES,), (wid >> 2) * M, jnp.int32)
    lax.fori_loop(0, 64, lambda i, c: chunk_body(
        i, LVL_OFF[0], bb_main, None, 64.0, 24.0), 0)
    lax.fori_loop(0, 16, lambda i, c: chunk_body(
        i, LVL_OFF[1], bb_main, 64.0, 128.0, 48.0), 0)
    lax.fori_loop(0, 4, lambda i, c: chunk_body(
        i, LVL_OFF[2], bb_main, 128.0, 256.0, 96.0), 0)
    lax.fori_loop(0, 1, lambda i, c: chunk_body(
        i, LVL_OFF[3], bb_main, 256.0, 512.0, 192.0), 0)

    @pl.when(wid < 8)
    def _l4():
        lax.fori_loop(0, 1, lambda i, c: chunk_body(
            i, LVL_OFF[4], jnp.full((LANES,), wid * M, jnp.int32),
            512.0, None, 384.0), 0)

    for li, (k_h, r_h, n_h) in enumerate(
            ((k0_h, r0_h, n0_h), (k1_h, r1_h, n1_h), (k2_h, r2_h, n2_h),
             (k3_h, r3_h, n3_h))):
        npx, soff = LVL_PX[li], LVL_OFF[li]
        pltpu.sync_copy(ksv.at[pl.ds(soff, npx)],
                        k_h.at[pl.ds(wid * npx, npx)])
        pltpu.sync_copy(nsv.at[pl.ds(soff, npx)],
                        n_h.at[pl.ds(wid * npx, npx)])
        pltpu.sync_copy(rgv.at[pl.ds(soff * 4, npx * 4)],
                        r_h.at[pl.ds(wid * npx * 4, npx * 4)])

    @pl.when(wid < 8)
    def _l4out():
        npx, soff = LVL_PX[4], LVL_OFF[4]
        pltpu.sync_copy(ksv.at[pl.ds(soff, npx)],
                        k4_h.at[pl.ds(wid * npx, npx)])
        pltpu.sync_copy(nsv.at[pl.ds(soff, npx)],
                        n4_h.at[pl.ds(wid * npx, npx)])
        pltpu.sync_copy(rgv.at[pl.ds(soff * 4, npx * 4)],
                        r4_h.at[pl.ds(wid * npx * 4, npx * 4)])


@jax.jit
def _targets(bbox_gt, cls_gt):
    bbx = bbox_gt.reshape(-1)
    cls = cls_gt.astype(jnp.int32).reshape(-1)

    mesh = plsc.VectorSubcoreMesh(core_axis_name="c", subcore_axis_name="s")
    f32 = jnp.float32
    i32 = jnp.int32
    out_type = (
        tuple(jax.ShapeDtypeStruct((B * h * w,), i32) for h, w in HW)
        + tuple(jax.ShapeDtypeStruct((B * h * w * 4,), f32) for h, w in HW)
        + tuple(jax.ShapeDtypeStruct((B * h * w,), f32) for h, w in HW)
    )
    scratch = [
        pltpu.VMEM((B * M * 4,), f32),
        pltpu.VMEM((B * M,), i32),
        pltpu.VMEM((B * M,), f32),
        pltpu.VMEM((B * M,), f32),
        pltpu.VMEM((B * M,), f32),
        pltpu.VMEM((B * M,), f32),
        pltpu.VMEM((B * M,), f32),
        pltpu.VMEM((B * M,), f32),
        pltpu.VMEM((PW,), f32),
        pltpu.VMEM((PW,), f32),
        pltpu.VMEM((PW,), i32),
        pltpu.VMEM((PW,), f32),
        pltpu.VMEM((PW * 4,), f32),
    ]
    run = pl.kernel(_sc_body, out_type=out_type, mesh=mesh,
                    scratch_types=scratch,
                    compiler_params=pltpu.CompilerParams(
                        needs_layout_passes=False))
    outs = run(bbx, cls, jnp.asarray(_XF), jnp.asarray(_YF))
    ks, rs, ns = outs[0:5], outs[5:10], outs[10:15]
    cls_target = tuple(k.reshape(B, h * w, 1) for k, (h, w) in zip(ks, HW))
    reg_target = tuple(r.reshape(B, h * w, 4) for r, (h, w) in zip(rs, HW))
    cnt_target = tuple(n.reshape(B, h * w, 1) for n, (h, w) in zip(ns, HW))
    return cls_target, reg_target, cnt_target


def kernel(cls_head_0, cls_head_1, cls_head_2, cls_head_3, cls_head_4,
           reg_head_0, reg_head_1, reg_head_2, reg_head_3, reg_head_4,
           cnt_head_0, cnt_head_1, cnt_head_2, cnt_head_3, cnt_head_4,
           bbox_gt, cls_gt):
    del cls_head_0, cls_head_1, cls_head_2, cls_head_3, cls_head_4
    del reg_head_0, reg_head_1, reg_head_2, reg_head_3, reg_head_4
    del cnt_head_0, cnt_head_1, cnt_head_2, cnt_head_3, cnt_head_4
    return _targets(bbox_gt, cls_gt)


# speedup vs baseline: 1.5379x; 1.5379x over previous
import jax
import jax.numpy as jnp
import numpy as np
from jax import lax
from jax.experimental import pallas as pl
from jax.experimental.pallas import tpu as pltpu
from jax.experimental.pallas import tpu_sc as plsc

STRIDES = (8, 16, 32, 64, 128)
HW = ((64, 64), (32, 32), (16, 16), (8, 8), (4, 4))
B = 8
M = 50
BIG = 99999999.0

NW = 32
LANES = 16
PW = 1376

LVL_PX = (1024, 256, 64, 16, 16)
LVL_OFF = (0, 1024, 1280, 1344, 1360)


def _build_coords():
    xf = np.zeros(NW * PW, np.float32)
    yf = np.zeros(NW * PW, np.float32)
    for w in range(NW):
        base = w * PW
        for li, (h, wd) in enumerate(HW):
            s = STRIDES[li]
            npx = LVL_PX[li]
            if li == 4 and w >= 8:
                continue
            f = w * npx + np.arange(npx)
            p = f % (h * wd)
            xf[base + LVL_OFF[li]: base + LVL_OFF[li] + npx] = \
                (p % wd) * s + s // 2
            yf[base + LVL_OFF[li]: base + LVL_OFF[li] + npx] = \
                (p // wd) * s + s // 2
    return xf, yf


_XF, _YF = _build_coords()


def _sqrt16(x):
    xi = lax.bitcast_convert_type(x, jnp.int32)
    yi = jnp.int32(0x5F3759DF) - (xi >> 1)
    y = lax.bitcast_convert_type(yi, jnp.float32)
    hx = x * 0.5
    for _ in range(4):
        y = y * (1.5 - hx * y * y)
    return x * y


def _sc_body(bbx_h, cls_h, xf_h, yf_h,
             k0_h, k1_h, k2_h, k3_h, k4_h,
             r0_h, r1_h, r2_h, r3_h, r4_h,
             n0_h, n1_h, n2_h, n3_h, n4_h,
             bbxv, clsv, x0v, y0v, x1v, y1v, sxv, syv,
             xv, yv, ksv, nsv, rg0, rg1, rg2, rg3, rg4):
    wid = lax.axis_index("s") * 2 + lax.axis_index("c")
    poff = wid * PW
    pltpu.sync_copy(bbx_h, bbxv)
    pltpu.sync_copy(cls_h, clsv)
    pltpu.sync_copy(xf_h.at[pl.ds(poff, PW)], xv)
    pltpu.sync_copy(yf_h.at[pl.ds(poff, PW)], yv)

    iota = lax.iota(jnp.int32, LANES)
    iota4 = iota * 4
    for v in range(B * M // LANES):
        idx = iota4 + (64 * v)
        gx0 = plsc.load_gather(bbxv, (idx,))
        gy0 = plsc.load_gather(bbxv, (idx + 1,))
        gx1 = plsc.load_gather(bbxv, (idx + 2,))
        gy1 = plsc.load_gather(bbxv, (idx + 3,))
        sl = pl.ds(v * LANES, LANES)
        x0v[sl] = gx0
        y0v[sl] = gy0
        x1v[sl] = gx1
        y1v[sl] = gy1
        sxv[sl] = gx0 + gx1
        syv[sl] = gy0 + gy1

    def chunk_body(i, soff, bb, wlo, whi, rad2, reg_store):
        s = soff + i * LANES
        px = xv[pl.ds(s, LANES)]
        py = yv[pl.ds(s, LANES)]
        x2 = px + px
        y2 = py + py
        best_a = jnp.full((LANES,), BIG, jnp.float32)
        best_m = jnp.zeros((LANES,), jnp.int32)
        anyp = jnp.zeros((LANES,), jnp.bool_)
        for m in range(M):
            mi = bb + m
            bx0 = plsc.load_gather(x0v, (mi,))
            by0 = plsc.load_gather(y0v, (mi,))
            bx1 = plsc.load_gather(x1v, (mi,))
            by1 = plsc.load_gather(y1v, (mi,))
            bsx = plsc.load_gather(sxv, (mi,))
            bsy = plsc.load_gather(syv, (mi,))
            l = px - bx0
            t = py - by0
            r = bx1 - px
            b2 = by1 - py
            dmin = jnp.minimum(jnp.minimum(l, t), jnp.minimum(r, b2))
            dmax = jnp.maximum(jnp.maximum(l, t), jnp.maximum(r, b2))
            dc = jnp.maximum(jnp.abs(x2 - bsx), jnp.abs(y2 - bsy))
            pos = (dmin > 0.0) & (dc < rad2)
            if whi is not None:
                pos = pos & (dmax <= whi)
            if wlo is not None:
                pos = pos & (dmax >= wlo)
            area = (l + r) * (t + b2)
            am = jnp.where(pos, area, BIG)
            better = am < best_a
            best_a = jnp.where(better, am, best_a)
            best_m = jnp.where(better, jnp.int32(m), best_m)
            anyp = anyp | pos
        gi = bb + best_m
        gx0 = plsc.load_gather(x0v, (gi,))
        gy0 = plsc.load_gather(y0v, (gi,))
        gx1 = plsc.load_gather(x1v, (gi,))
        gy1 = plsc.load_gather(y1v, (gi,))
        gcls = plsc.load_gather(clsv, (gi,))
        l = px - gx0
        t = py - gy0
        r = gx1 - px
        b2 = gy1 - py
        lrmin = jnp.minimum(l, r)
        lrmax = jnp.maximum(l, r)
        tbmin = jnp.minimum(t, b2)
        tbmax = jnp.maximum(t, b2)
        ratio = jnp.maximum(lrmin * tbmin / (lrmax * tbmax + 1e-10), 0.0)
        cnt = _sqrt16(ratio + 1e-12)
        neg1 = jnp.full((LANES,), -1.0, jnp.float32)
        reg_store(i, jnp.where(anyp, l, neg1), jnp.where(anyp, t, neg1),
                  jnp.where(anyp, r, neg1), jnp.where(anyp, b2, neg1))
        nsv[pl.ds(s, LANES)] = jnp.where(anyp, cnt, neg1)
        ksv[pl.ds(s, LANES)] = jnp.where(
            anyp, gcls, jnp.zeros((LANES,), jnp.int32))
        return 0

    def tiled_store(rgbuf):
        def store(i, lv, tv, rv, bv):
            off = (i >> 3) * 512 + (i & 7) * LANES
            rgbuf[pl.ds(off, LANES)] = lv
            rgbuf[pl.ds(off + 128, LANES)] = tv
            rgbuf[pl.ds(off + 256, LANES)] = rv
            rgbuf[pl.ds(off + 384, LANES)] = bv
        return store

    def plane_store(rgbuf, plane):
        def store(i, lv, tv, rv, bv):
            off = i * LANES
            rgbuf[pl.ds(off, LANES)] = lv
            rgbuf[pl.ds(off + plane, LANES)] = tv
            rgbuf[pl.ds(off + 2 * plane, LANES)] = rv
            rgbuf[pl.ds(off + 3 * plane, LANES)] = bv
        return store

    bb_main = jnp.full((LANES,), (wid >> 2) * M, jnp.int32)
    lax.fori_loop(0, 64, lambda i, c: chunk_body(
        i, LVL_OFF[0], bb_main, None, 64.0, 24.0, tiled_store(rg0)), 0)
    lax.fori_loop(0, 16, lambda i, c: chunk_body(
        i, LVL_OFF[1], bb_main, 64.0, 128.0, 48.0, tiled_store(rg1)), 0)
    lax.fori_loop(0, 4, lambda i, c: chunk_body(
        i, LVL_OFF[2], bb_main, 128.0, 256.0, 96.0,
        plane_store(rg2, 128)), 0)
    lax.fori_loop(0, 1, lambda i, c: chunk_body(
        i, LVL_OFF[3], bb_main, 256.0, 512.0, 192.0,
        plane_store(rg3, LANES)), 0)

    @pl.when(wid < 8)
    def _l4():
        lax.fori_loop(0, 1, lambda i, c: chunk_body(
            i, LVL_OFF[4], jnp.full((LANES,), wid * M, jnp.int32),
            512.0, None, 384.0, plane_store(rg4, LANES)), 0)

    b = wid >> 2
    w4 = wid & 3
    for li, (k_h, n_h) in enumerate(((k0_h, n0_h), (k1_h, n1_h),
                                     (k2_h, n2_h), (k3_h, n3_h))):
        npx, soff = LVL_PX[li], LVL_OFF[li]
        pltpu.sync_copy(ksv.at[pl.ds(soff, npx)],
                        k_h.at[pl.ds(wid * npx, npx)])
        pltpu.sync_copy(nsv.at[pl.ds(soff, npx)],
                        n_h.at[pl.ds(wid * npx, npx)])
    pltpu.sync_copy(rg0, r0_h.at[pl.ds(wid * 4096, 4096)])
    pltpu.sync_copy(rg1, r1_h.at[pl.ds(wid * 1024, 1024)])
    t2 = w4 >> 1
    l2 = (w4 & 1) * 64
    for c in range(4):
        pltpu.sync_copy(rg2.at[pl.ds(c * 128, 64)],
                        r2_h.at[pl.ds(b * 1024 + t2 * 512 + c * 128 + l2, 64)])
        pltpu.sync_copy(rg3.at[pl.ds(c * LANES, LANES)],
                        r3_h.at[pl.ds(b * 512 + c * 128 + w4 * LANES, LANES)])

    @pl.when(wid < 8)
    def _l4out():
        npx, soff = LVL_PX[4], LVL_OFF[4]
        pltpu.sync_copy(ksv.at[pl.ds(soff, npx)],
                        k4_h.at[pl.ds(wid * npx, npx)])
        pltpu.sync_copy(nsv.at[pl.ds(soff, npx)],
                        n4_h.at[pl.ds(wid * npx, npx)])
        for c in range(4):
            pltpu.sync_copy(rg4.at[pl.ds(c * LANES, LANES)],
                            r4_h.at[pl.ds(wid * 512 + c * 128, LANES)])


@jax.jit
def _targets(bbox_gt, cls_gt):
    bbx = bbox_gt.reshape(-1)
    cls = cls_gt.astype(jnp.int32).reshape(-1)

    mesh = plsc.VectorSubcoreMesh(core_axis_name="c", subcore_axis_name="s")
    f32 = jnp.float32
    i32 = jnp.int32
    reg_words = (B * 4096 * 4, B * 1024 * 4, B * 256 * 4, B * 512, B * 512)
    out_type = (
        tuple(jax.ShapeDtypeStruct((B * h * w,), i32) for h, w in HW)
        + tuple(jax.ShapeDtypeStruct((n,), f32) for n in reg_words)
        + tuple(jax.ShapeDtypeStruct((B * h * w,), f32) for h, w in HW)
    )
    scratch = [
        pltpu.VMEM((B * M * 4,), f32),
        pltpu.VMEM((B * M,), i32),
        pltpu.VMEM((B * M,), f32),
        pltpu.VMEM((B * M,), f32),
        pltpu.VMEM((B * M,), f32),
        pltpu.VMEM((B * M,), f32),
        pltpu.VMEM((B * M,), f32),
        pltpu.VMEM((B * M,), f32),
        pltpu.VMEM((PW,), f32),
        pltpu.VMEM((PW,), f32),
        pltpu.VMEM((PW,), i32),
        pltpu.VMEM((PW,), f32),
        pltpu.VMEM((4096,), f32),
        pltpu.VMEM((1024,), f32),
        pltpu.VMEM((512,), f32),
        pltpu.VMEM((64,), f32),
        pltpu.VMEM((64,), f32),
    ]
    run = pl.kernel(_sc_body, out_type=out_type, mesh=mesh,
                    scratch_types=scratch,
                    compiler_params=pltpu.CompilerParams(
                        needs_layout_passes=False))
    outs = run(bbx, cls, jnp.asarray(_XF), jnp.asarray(_YF))
    ks, rs, ns = outs[0:5], outs[5:10], outs[10:15]
    cls_target = tuple(k.reshape(B, h * w, 1) for k, (h, w) in zip(ks, HW))
    cnt_target = tuple(n.reshape(B, h * w, 1) for n, (h, w) in zip(ns, HW))
    reg_target = []
    for li, (h, w) in enumerate(HW):
        hw = h * w
        nt = max(hw // 128, 1)
        r = rs[li].reshape(B, nt, 4, 128).transpose(0, 1, 3, 2)
        r = r.reshape(B, nt * 128, 4)
        if hw < 128:
            r = lax.slice(r, (0, 0, 0), (B, hw, 4))
        reg_target.append(r)
    return cls_target, tuple(reg_target), cnt_target


def kernel(cls_head_0, cls_head_1, cls_head_2, cls_head_3, cls_head_4,
           reg_head_0, reg_head_1, reg_head_2, reg_head_3, reg_head_4,
           cnt_head_0, cnt_head_1, cnt_head_2, cnt_head_3, cnt_head_4,
           bbox_gt, cls_gt):
    del cls_head_0, cls_head_1, cls_head_2, cls_head_3, cls_head_4
    del reg_head_0, reg_head_1, reg_head_2, reg_head_3, reg_head_4
    del cnt_head_0, cnt_head_1, cnt_head_2, cnt_head_3, cnt_head_4
    return _targets(bbox_gt, cls_gt)


# speedup vs baseline: 1.7690x; 1.1503x over previous
import jax
import jax.numpy as jnp
from jax import lax
from jax.experimental import pallas as pl
from jax.experimental.pallas import tpu as pltpu
from jax.experimental.pallas import tpu_sc as plsc

HW = ((64, 64), (32, 32), (16, 16), (8, 8), (4, 4))
B = 8
M = 50
UNROLL = 5
BIG = 99999999.0

NW = 32
LANES = 16
PW = 1376
LVL_PX = (1024, 256, 64, 16, 16)
LVL_OFF = (0, 1024, 1280, 1344, 1360)


def _sqrt16(x):
    xi = lax.bitcast_convert_type(x, jnp.int32)
    yi = jnp.int32(0x5F3759DF) - (xi >> 1)
    y = lax.bitcast_convert_type(yi, jnp.float32)
    hx = x * 0.5
    for _ in range(4):
        y = y * (1.5 - hx * y * y)
    return x * y


def _sc_body(bbx_h, cls_h,
             k0_h, k1_h, k2_h, k3_h, k4_h,
             r0_h, r1_h, r2_h, r3_h, r4_h,
             n0_h, n1_h, n2_h, n3_h, n4_h,
             bbxv, clsv, x0v, y0v, x1v, y1v, sxv, syv,
             ksv, nsv, rg0, rg1, rg2, rg3, rg4):
    wid = lax.axis_index("s") * 2 + lax.axis_index("c")
    pltpu.sync_copy(bbx_h, bbxv)
    pltpu.sync_copy(cls_h, clsv)

    iota = lax.iota(jnp.int32, LANES)
    iota4 = iota * 4
    iotaf = iota.astype(jnp.float32)

    def planar(v, c):
        idx = iota4 + v * 64
        gx0 = plsc.load_gather(bbxv, (idx,))
        gy0 = plsc.load_gather(bbxv, (idx + 1,))
        gx1 = plsc.load_gather(bbxv, (idx + 2,))
        gy1 = plsc.load_gather(bbxv, (idx + 3,))
        sl = pl.ds(v * LANES, LANES)
        x0v[sl] = gx0
        y0v[sl] = gy0
        x1v[sl] = gx1
        y1v[sl] = gy1
        sxv[sl] = gx0 + gx1
        syv[sl] = gy0 + gy1
        return 0

    lax.fori_loop(0, B * M // LANES, planar, 0)

    def chunk_body(i, soff, bb, px, py, wlo, whi, rad2, reg_store):
        s = soff + i * LANES
        x2 = px + px
        y2 = py + py

        def box_group(it, carry):
            best_a, best_mi, anyp, mi = carry
            for _ in range(UNROLL):
                bx0 = plsc.load_gather(x0v, (mi,))
                by0 = plsc.load_gather(y0v, (mi,))
                bx1 = plsc.load_gather(x1v, (mi,))
                by1 = plsc.load_gather(y1v, (mi,))
                bsx = plsc.load_gather(sxv, (mi,))
                bsy = plsc.load_gather(syv, (mi,))
                l = px - bx0
                t = py - by0
                r = bx1 - px
                b2 = by1 - py
                dmin = jnp.minimum(jnp.minimum(l, t), jnp.minimum(r, b2))
                dmax = jnp.maximum(jnp.maximum(l, t), jnp.maximum(r, b2))
                dc = jnp.maximum(jnp.abs(x2 - bsx), jnp.abs(y2 - bsy))
                pos = (dmin > 0.0) & (dc < rad2)
                if whi is not None:
                    pos = pos & (dmax <= whi)
                if wlo is not None:
                    pos = pos & (dmax >= wlo)
                area = (l + r) * (t + b2)
                am = jnp.where(pos, area, BIG)
                better = am < best_a
                best_a = jnp.where(better, am, best_a)
                best_mi = jnp.where(better, mi, best_mi)
                anyp = anyp | pos
                mi = mi + 1
            return best_a, best_mi, anyp, mi

        init = (jnp.full((LANES,), BIG, jnp.float32), bb,
                jnp.zeros((LANES,), jnp.bool_), bb)
        _, gi, anyp, _ = lax.fori_loop(0, M // UNROLL, box_group, init)
        gx0 = plsc.load_gather(x0v, (gi,))
        gy0 = plsc.load_gather(y0v, (gi,))
        gx1 = plsc.load_gather(x1v, (gi,))
        gy1 = plsc.load_gather(y1v, (gi,))
        gcls = plsc.load_gather(clsv, (gi,))
        l = px - gx0
        t = py - gy0
        r = gx1 - px
        b2 = gy1 - py
        lrmin = jnp.minimum(l, r)
        lrmax = jnp.maximum(l, r)
        tbmin = jnp.minimum(t, b2)
        tbmax = jnp.maximum(t, b2)
        ratio = jnp.maximum(lrmin * tbmin / (lrmax * tbmax + 1e-10), 0.0)
        cnt = _sqrt16(ratio + 1e-12)
        neg1 = jnp.full((LANES,), -1.0, jnp.float32)
        reg_store(i, jnp.where(anyp, l, neg1), jnp.where(anyp, t, neg1),
                  jnp.where(anyp, r, neg1), jnp.where(anyp, b2, neg1))
        nsv[pl.ds(s, LANES)] = jnp.where(anyp, cnt, neg1)
        ksv[pl.ds(s, LANES)] = jnp.where(
            anyp, gcls, jnp.zeros((LANES,), jnp.int32))
        return 0

    def tiled_store(rgbuf):
        def store(i, lv, tv, rv, bv):
            off = (i >> 3) * 512 + (i & 7) * LANES
            rgbuf[pl.ds(off, LANES)] = lv
            rgbuf[pl.ds(off + 128, LANES)] = tv
            rgbuf[pl.ds(off + 256, LANES)] = rv
            rgbuf[pl.ds(off + 384, LANES)] = bv
        return store

    def plane_store(rgbuf, plane):
        def store(i, lv, tv, rv, bv):
            off = i * LANES
            rgbuf[pl.ds(off, LANES)] = lv
            rgbuf[pl.ds(off + plane, LANES)] = tv
            rgbuf[pl.ds(off + 2 * plane, LANES)] = rv
            rgbuf[pl.ds(off + 3 * plane, LANES)] = bv
        return store

    bb_main = jnp.full((LANES,), (wid >> 2) * M, jnp.int32)

    def fxy(row_scalar, xbase_scalar, xstep):
        px = iotaf * float(xstep) + xbase_scalar.astype(jnp.float32)
        py = jnp.full((LANES,), row_scalar, jnp.int32).astype(jnp.float32)
        return px, py

    def l0(i, c):
        px, py = fxy((wid & 3) * 128 + (i >> 2) * 8 + 4, (i & 3) * 128 + 4, 8)
        return chunk_body(i, LVL_OFF[0], bb_main, px, py,
                          None, 64.0, 24.0, tiled_store(rg0))

    def l1(i, c):
        px, py = fxy((wid & 3) * 128 + (i >> 1) * 16 + 8, (i & 1) * 256 + 8,
                     16)
        return chunk_body(i, LVL_OFF[1], bb_main, px, py,
                          64.0, 128.0, 48.0, tiled_store(rg1))

    def l2(i, c):
        px, py = fxy((wid & 3) * 128 + i * 32 + 16, i * 0 + 16, 32)
        return chunk_body(i, LVL_OFF[2], bb_main, px, py,
                          128.0, 256.0, 96.0, plane_store(rg2, 128))

    def l3(i, c):
        px = ((iota & 7) * 64 + 32).astype(jnp.float32)
        py = ((iota >> 3) * 64 + ((wid & 3) * 128 + 32)).astype(jnp.float32)
        return chunk_body(i, LVL_OFF[3], bb_main, px, py,
                          256.0, 512.0, 192.0, plane_store(rg3, LANES))

    lax.fori_loop(0, 64, l0, 0)
    lax.fori_loop(0, 16, l1, 0)
    lax.fori_loop(0, 4, l2, 0)
    lax.fori_loop(0, 1, l3, 0)

    @pl.when(wid < 8)
    def _l4():
        def l4(i, c):
            px = ((iota & 3) * 128 + 64).astype(jnp.float32)
            py = ((iota >> 2) * 128 + 64).astype(jnp.float32)
            return chunk_body(i, LVL_OFF[4],
                              jnp.full((LANES,), wid * M, jnp.int32),
                              px, py, 512.0, None, 384.0,
                              plane_store(rg4, LANES))
        lax.fori_loop(0, 1, l4, 0)

    b = wid >> 2
    w4 = wid & 3
    for li, (k_h, n_h) in enumerate(((k0_h, n0_h), (k1_h, n1_h),
                                     (k2_h, n2_h), (k3_h, n3_h))):
        npx, soff = LVL_PX[li], LVL_OFF[li]
        pltpu.sync_copy(ksv.at[pl.ds(soff, npx)],
                        k_h.at[pl.ds(wid * npx, npx)])
        pltpu.sync_copy(nsv.at[pl.ds(soff, npx)],
                        n_h.at[pl.ds(wid * npx, npx)])
    pltpu.sync_copy(rg0, r0_h.at[pl.ds(wid * 4096, 4096)])
    pltpu.sync_copy(rg1, r1_h.at[pl.ds(wid * 1024, 1024)])
    t2 = w4 >> 1
    l2o = (w4 & 1) * 64
    for c in range(4):
        pltpu.sync_copy(rg2.at[pl.ds(c * 128, 64)],
                        r2_h.at[pl.ds(b * 1024 + t2 * 512 + c * 128 + l2o, 64)])
        pltpu.sync_copy(rg3.at[pl.ds(c * LANES, LANES)],
                        r3_h.at[pl.ds(b * 512 + c * 128 + w4 * LANES, LANES)])

    @pl.when(wid < 8)
    def _l4out():
        npx, soff = LVL_PX[4], LVL_OFF[4]
        pltpu.sync_copy(ksv.at[pl.ds(soff, npx)],
                        k4_h.at[pl.ds(wid * npx, npx)])
        pltpu.sync_copy(nsv.at[pl.ds(soff, npx)],
                        n4_h.at[pl.ds(wid * npx, npx)])
        for c in range(4):
            pltpu.sync_copy(rg4.at[pl.ds(c * LANES, LANES)],
                            r4_h.at[pl.ds(wid * 512 + c * 128, LANES)])


@jax.jit
def _targets(bbox_gt, cls_gt):
    bbx = bbox_gt.reshape(-1)
    cls = cls_gt.astype(jnp.int32).reshape(-1)

    mesh = plsc.VectorSubcoreMesh(core_axis_name="c", subcore_axis_name="s")
    f32 = jnp.float32
    i32 = jnp.int32
    reg_words = (B * 4096 * 4, B * 1024 * 4, B * 256 * 4, B * 512, B * 512)
    out_type = (
        tuple(jax.ShapeDtypeStruct((B * h * w,), i32) for h, w in HW)
        + tuple(jax.ShapeDtypeStruct((n,), f32) for n in reg_words)
        + tuple(jax.ShapeDtypeStruct((B * h * w,), f32) for h, w in HW)
    )
    scratch = [
        pltpu.VMEM((B * M * 4,), f32),
        pltpu.VMEM((B * M,), i32),
        pltpu.VMEM((B * M,), f32),
        pltpu.VMEM((B * M,), f32),
        pltpu.VMEM((B * M,), f32),
        pltpu.VMEM((B * M,), f32),
        pltpu.VMEM((B * M,), f32),
        pltpu.VMEM((B * M,), f32),
        pltpu.VMEM((PW,), i32),
        pltpu.VMEM((PW,), f32),
        pltpu.VMEM((4096,), f32),
        pltpu.VMEM((1024,), f32),
        pltpu.VMEM((512,), f32),
        pltpu.VMEM((64,), f32),
        pltpu.VMEM((64,), f32),
    ]
    run = pl.kernel(_sc_body, out_type=out_type, mesh=mesh,
                    scratch_types=scratch,
                    compiler_params=pltpu.CompilerParams(
                        needs_layout_passes=False))
    outs = run(bbx, cls)
    ks, rs, ns = outs[0:5], outs[5:10], outs[10:15]
    cls_target = tuple(k.reshape(B, h * w, 1) for k, (h, w) in zip(ks, HW))
    cnt_target = tuple(n.reshape(B, h * w, 1) for n, (h, w) in zip(ns, HW))
    reg_target = []
    for li, (h, w) in enumerate(HW):
        hw = h * w
        nt = max(hw // 128, 1)
        r = rs[li].reshape(B, nt, 4, 128).transpose(0, 1, 3, 2)
        r = r.reshape(B, nt * 128, 4)
        if hw < 128:
            r = lax.slice(r, (0, 0, 0), (B, hw, 4))
        reg_target.append(r)
    return cls_target, tuple(reg_target), cnt_target


def kernel(cls_head_0, cls_head_1, cls_head_2, cls_head_3, cls_head_4,
           reg_head_0, reg_head_1, reg_head_2, reg_head_3, reg_head_4,
           cnt_head_0, cnt_head_1, cnt_head_2, cnt_head_3, cnt_head_4,
           bbox_gt, cls_gt):
    del cls_head_0, cls_head_1, cls_head_2, cls_head_3, cls_head_4
    del reg_head_0, reg_head_1, reg_head_2, reg_head_3, reg_head_4
    del cnt_head_0, cnt_head_1, cnt_head_2, cnt_head_3, cnt_head_4
    return _targets(bbox_gt, cls_gt)


# speedup vs baseline: 1.8807x; 1.0631x over previous
import jax
import jax.numpy as jnp
from jax import lax
from jax.experimental import pallas as pl
from jax.experimental.pallas import tpu as pltpu
from jax.experimental.pallas import tpu_sc as plsc

HW = ((64, 64), (32, 32), (16, 16), (8, 8), (4, 4))
B = 8
M = 50
UNROLL = 5
BIG = 99999999.0

NW = 32
LANES = 16
PW = 1376
LVL_PX = (1024, 256, 64, 16, 16)
LVL_OFF = (0, 1024, 1280, 1344, 1360)


def _sqrt16(x):
    xi = lax.bitcast_convert_type(x, jnp.int32)
    yi = jnp.int32(0x5F3759DF) - (xi >> 1)
    y = lax.bitcast_convert_type(yi, jnp.float32)
    hx = x * 0.5
    for _ in range(4):
        y = y * (1.5 - hx * y * y)
    return x * y


def _sc_body(bbx_h, cls_h,
             k0_h, k1_h, k2_h, k3_h, k4_h,
             r0_h, r1_h, r2_h, r3_h, r4_h,
             n0_h, n1_h, n2_h, n3_h, n4_h,
             bbxv, clsv, x0v, y0v, x1v, y1v, sxv, syv, arv,
             ksv, nsv, rg0, rg1, rg2, rg3, rg4):
    wid = lax.axis_index("s") * 2 + lax.axis_index("c")
    pltpu.sync_copy(bbx_h, bbxv)
    pltpu.sync_copy(cls_h, clsv)

    iota = lax.iota(jnp.int32, LANES)
    iota4 = iota * 4
    iotaf = iota.astype(jnp.float32)

    def planar(v, c):
        idx = iota4 + v * 64
        gx0 = plsc.load_gather(bbxv, (idx,))
        gy0 = plsc.load_gather(bbxv, (idx + 1,))
        gx1 = plsc.load_gather(bbxv, (idx + 2,))
        gy1 = plsc.load_gather(bbxv, (idx + 3,))
        sl = pl.ds(v * LANES, LANES)
        x0v[sl] = gx0
        y0v[sl] = gy0
        x1v[sl] = gx1
        y1v[sl] = gy1
        sxv[sl] = gx0 + gx1
        syv[sl] = gy0 + gy1
        arv[sl] = (gx1 - gx0) * (gy1 - gy0)
        return 0

    lax.fori_loop(0, B * M // LANES, planar, 0)

    def chunk_body(i, soff, bb, px, py, wlo, whi, rad2, reg_store):
        s = soff + i * LANES
        x2 = px + px
        y2 = py + py

        def box_group(it, carry):
            best_a, best_mi, mi = carry
            for _ in range(UNROLL):
                bx0 = plsc.load_gather(x0v, (mi,))
                by0 = plsc.load_gather(y0v, (mi,))
                bx1 = plsc.load_gather(x1v, (mi,))
                by1 = plsc.load_gather(y1v, (mi,))
                bsx = plsc.load_gather(sxv, (mi,))
                bsy = plsc.load_gather(syv, (mi,))
                area = plsc.load_gather(arv, (mi,))
                l = px - bx0
                t = py - by0
                r = bx1 - px
                b2 = by1 - py
                dmin = jnp.minimum(jnp.minimum(l, t), jnp.minimum(r, b2))
                dmax = jnp.maximum(jnp.maximum(l, t), jnp.maximum(r, b2))
                dc = jnp.maximum(jnp.abs(x2 - bsx), jnp.abs(y2 - bsy))
                pos = (dmin > 0.0) & (dc < rad2)
                if whi is not None:
                    pos = pos & (dmax <= whi)
                if wlo is not None:
                    pos = pos & (dmax >= wlo)
                better = pos & (area < best_a)
                best_a = jnp.where(better, area, best_a)
                best_mi = jnp.where(better, mi, best_mi)
                mi = mi + 1
            return best_a, best_mi, mi

        init = (jnp.full((LANES,), BIG, jnp.float32), bb, bb)
        best_a, gi, _ = lax.fori_loop(0, M // UNROLL, box_group, init)
        anyp = best_a < BIG
        gx0 = plsc.load_gather(x0v, (gi,))
        gy0 = plsc.load_gather(y0v, (gi,))
        gx1 = plsc.load_gather(x1v, (gi,))
        gy1 = plsc.load_gather(y1v, (gi,))
        gcls = plsc.load_gather(clsv, (gi,))
        l = px - gx0
        t = py - gy0
        r = gx1 - px
        b2 = gy1 - py
        lrmin = jnp.minimum(l, r)
        lrmax = jnp.maximum(l, r)
        tbmin = jnp.minimum(t, b2)
        tbmax = jnp.maximum(t, b2)
        ratio = jnp.maximum(lrmin * tbmin / (lrmax * tbmax + 1e-10), 0.0)
        cnt = _sqrt16(ratio + 1e-12)
        neg1 = jnp.full((LANES,), -1.0, jnp.float32)
        reg_store(i, jnp.where(anyp, l, neg1), jnp.where(anyp, t, neg1),
                  jnp.where(anyp, r, neg1), jnp.where(anyp, b2, neg1))
        nsv[pl.ds(s, LANES)] = jnp.where(anyp, cnt, neg1)
        ksv[pl.ds(s, LANES)] = jnp.where(
            anyp, gcls, jnp.zeros((LANES,), jnp.int32))
        return 0

    def tiled_store(rgbuf):
        def store(i, lv, tv, rv, bv):
            off = (i >> 3) * 512 + (i & 7) * LANES
            rgbuf[pl.ds(off, LANES)] = lv
            rgbuf[pl.ds(off + 128, LANES)] = tv
            rgbuf[pl.ds(off + 256, LANES)] = rv
            rgbuf[pl.ds(off + 384, LANES)] = bv
        return store

    def plane_store(rgbuf, plane):
        def store(i, lv, tv, rv, bv):
            off = i * LANES
            rgbuf[pl.ds(off, LANES)] = lv
            rgbuf[pl.ds(off + plane, LANES)] = tv
            rgbuf[pl.ds(off + 2 * plane, LANES)] = rv
            rgbuf[pl.ds(off + 3 * plane, LANES)] = bv
        return store

    bb_main = jnp.full((LANES,), (wid >> 2) * M, jnp.int32)

    def fxy(row_scalar, xbase_scalar, xstep):
        px = iotaf * float(xstep) + xbase_scalar.astype(jnp.float32)
        py = jnp.full((LANES,), row_scalar, jnp.int32).astype(jnp.float32)
        return px, py

    def l0(i, c):
        px, py = fxy((wid & 3) * 128 + (i >> 2) * 8 + 4, (i & 3) * 128 + 4, 8)
        return chunk_body(i, LVL_OFF[0], bb_main, px, py,
                          None, 64.0, 24.0, tiled_store(rg0))

    def l1(i, c):
        px, py = fxy((wid & 3) * 128 + (i >> 1) * 16 + 8, (i & 1) * 256 + 8,
                     16)
        return chunk_body(i, LVL_OFF[1], bb_main, px, py,
                          64.0, 128.0, 48.0, tiled_store(rg1))

    def l2(i, c):
        px, py = fxy((wid & 3) * 128 + i * 32 + 16, i * 0 + 16, 32)
        return chunk_body(i, LVL_OFF[2], bb_main, px, py,
                          128.0, 256.0, 96.0, plane_store(rg2, 128))

    def l3(i, c):
        px = ((iota & 7) * 64 + 32).astype(jnp.float32)
        py = ((iota >> 3) * 64 + ((wid & 3) * 128 + 32)).astype(jnp.float32)
        return chunk_body(i, LVL_OFF[3], bb_main, px, py,
                          256.0, 512.0, 192.0, plane_store(rg3, LANES))

    lax.fori_loop(0, 64, l0, 0)
    lax.fori_loop(0, 16, l1, 0)
    lax.fori_loop(0, 4, l2, 0)
    lax.fori_loop(0, 1, l3, 0)

    @pl.when(wid < 8)
    def _l4():
        def l4(i, c):
            px = ((iota & 3) * 128 + 64).astype(jnp.float32)
            py = ((iota >> 2) * 128 + 64).astype(jnp.float32)
            return chunk_body(i, LVL_OFF[4],
                              jnp.full((LANES,), wid * M, jnp.int32),
                              px, py, 512.0, None, 384.0,
                              plane_store(rg4, LANES))
        lax.fori_loop(0, 1, l4, 0)

    b = wid >> 2
    w4 = wid & 3
    for li, (k_h, n_h) in enumerate(((k0_h, n0_h), (k1_h, n1_h),
                                     (k2_h, n2_h), (k3_h, n3_h))):
        npx, soff = LVL_PX[li], LVL_OFF[li]
        pltpu.sync_copy(ksv.at[pl.ds(soff, npx)],
                        k_h.at[pl.ds(wid * npx, npx)])
        pltpu.sync_copy(nsv.at[pl.ds(soff, npx)],
                        n_h.at[pl.ds(wid * npx, npx)])
    pltpu.sync_copy(rg0, r0_h.at[pl.ds(wid * 4096, 4096)])
    pltpu.sync_copy(rg1, r1_h.at[pl.ds(wid * 1024, 1024)])
    t2 = w4 >> 1
    l2o = (w4 & 1) * 64
    for c in range(4):
        pltpu.sync_copy(rg2.at[pl.ds(c * 128, 64)],
                        r2_h.at[pl.ds(b * 1024 + t2 * 512 + c * 128 + l2o, 64)])
        pltpu.sync_copy(rg3.at[pl.ds(c * LANES, LANES)],
                        r3_h.at[pl.ds(b * 512 + c * 128 + w4 * LANES, LANES)])

    @pl.when(wid < 8)
    def _l4out():
        npx, soff = LVL_PX[4], LVL_OFF[4]
        pltpu.sync_copy(ksv.at[pl.ds(soff, npx)],
                        k4_h.at[pl.ds(wid * npx, npx)])
        pltpu.sync_copy(nsv.at[pl.ds(soff, npx)],
                        n4_h.at[pl.ds(wid * npx, npx)])
        for c in range(4):
            pltpu.sync_copy(rg4.at[pl.ds(c * LANES, LANES)],
                            r4_h.at[pl.ds(wid * 512 + c * 128, LANES)])


@jax.jit
def _targets(bbox_gt, cls_gt):
    bbx = bbox_gt.reshape(-1)
    cls = cls_gt.astype(jnp.int32).reshape(-1)

    mesh = plsc.VectorSubcoreMesh(core_axis_name="c", subcore_axis_name="s")
    f32 = jnp.float32
    i32 = jnp.int32
    reg_words = (B * 4096 * 4, B * 1024 * 4, B * 256 * 4, B * 512, B * 512)
    out_type = (
        tuple(jax.ShapeDtypeStruct((B * h * w,), i32) for h, w in HW)
        + tuple(jax.ShapeDtypeStruct((n,), f32) for n in reg_words)
        + tuple(jax.ShapeDtypeStruct((B * h * w,), f32) for h, w in HW)
    )
    scratch = [
        pltpu.VMEM((B * M * 4,), f32),
        pltpu.VMEM((B * M,), i32),
        pltpu.VMEM((B * M,), f32),
        pltpu.VMEM((B * M,), f32),
        pltpu.VMEM((B * M,), f32),
        pltpu.VMEM((B * M,), f32),
        pltpu.VMEM((B * M,), f32),
        pltpu.VMEM((B * M,), f32),
        pltpu.VMEM((B * M,), f32),
        pltpu.VMEM((PW,), i32),
        pltpu.VMEM((PW,), f32),
        pltpu.VMEM((4096,), f32),
        pltpu.VMEM((1024,), f32),
        pltpu.VMEM((512,), f32),
        pltpu.VMEM((64,), f32),
        pltpu.VMEM((64,), f32),
    ]
    run = pl.kernel(_sc_body, out_type=out_type, mesh=mesh,
                    scratch_types=scratch,
                    compiler_params=pltpu.CompilerParams(
                        needs_layout_passes=False))
    outs = run(bbx, cls)
    ks, rs, ns = outs[0:5], outs[5:10], outs[10:15]
    cls_target = tuple(k.reshape(B, h * w, 1) for k, (h, w) in zip(ks, HW))
    cnt_target = tuple(n.reshape(B, h * w, 1) for n, (h, w) in zip(ns, HW))
    reg_target = []
    for li, (h, w) in enumerate(HW):
        hw = h * w
        nt = max(hw // 128, 1)
        r = rs[li].reshape(B, nt, 4, 128).transpose(0, 1, 3, 2)
        r = r.reshape(B, nt * 128, 4)
        if hw < 128:
            r = lax.slice(r, (0, 0, 0), (B, hw, 4))
        reg_target.append(r)
    return cls_target, tuple(reg_target), cnt_target


def kernel(cls_head_0, cls_head_1, cls_head_2, cls_head_3, cls_head_4,
           reg_head_0, reg_head_1, reg_head_2, reg_head_3, reg_head_4,
           cnt_head_0, cnt_head_1, cnt_head_2, cnt_head_3, cnt_head_4,
           bbox_gt, cls_gt):
    del cls_head_0, cls_head_1, cls_head_2, cls_head_3, cls_head_4
    del reg_head_0, reg_head_1, reg_head_2, reg_head_3, reg_head_4
    del cnt_head_0, cnt_head_1, cnt_head_2, cnt_head_3, cnt_head_4
    return _targets(bbox_gt, cls_gt)
